# baseline (speedup 1.0000x reference)
import jax
import jax.numpy as jnp
from jax import lax
from jax.experimental import pallas as pl
from jax.experimental.pallas import tpu as pltpu
from jax.experimental.pallas import tpu_sc as plsc

N = 10000
E = 320000
F_IN = 128
F_HID = 128
F_OUT = 64

NC = 2
NS = 16
NW = NC * NS
EPT = E // NW
CH = 64
NFULL = EPT // CH
REM_E = EPT - NFULL * CH
NSETS = 4
NQUAD = NFULL // NSETS
NMAIN = NQUAD * NSETS
RPT = 624
R_TAIL0 = NS * RPT
R_TAIL = N - R_TAIL0
ZFULL = RPT // CH
ZREM = RPT - ZFULL * CH


def _make_sc_agg(D, with_deg):
    out_type = jax.ShapeDtypeStruct((NC, N, D), jnp.float32)
    if with_deg:
        out_type = (out_type, jax.ShapeDtypeStruct((NC * N,), jnp.float32))
    scratch = [
        pltpu.VMEM_SHARED((N, D), jnp.float32),
        pltpu.VMEM((EPT,), jnp.int32),
        pltpu.VMEM((REM_E,), jnp.int32),
    ]
    for _ in range(NSETS):
        scratch += [
            pltpu.VMEM((CH,), jnp.int32),
            pltpu.VMEM((CH, D), jnp.float32),
            pltpu.SemaphoreType.DMA,
            pltpu.SemaphoreType.DMA,
            pltpu.SemaphoreType.DMA,
            pltpu.SemaphoreType.DMA,
        ]
    if with_deg:
        scratch += [
            pltpu.VMEM_SHARED((N,), jnp.float32),
            pltpu.VMEM((CH,), jnp.float32),
            pltpu.VMEM((RPT,), jnp.float32),
        ]

    def body(feat, src, dst, *refs):
        if with_deg:
            out, deg_out = refs[0], refs[1]
            rest = refs[2:]
        else:
            out = refs[0]
            rest = refs[1:]
        acc_sh, src_all, dst_rem = rest[0], rest[1], rest[2]
        sets = [tuple(rest[3 + 6 * q: 3 + 6 * (q + 1)]) for q in range(NSETS)]
        if with_deg:
            deg_sh, ones_v, degrow_v = rest[3 + 6 * NSETS:]
        rows0 = sets[0][1]
        c = lax.axis_index("c")
        s = lax.axis_index("s")
        wid = c * NS + s
        r0 = pl.multiple_of(s * RPT, 8)
        ebase = pl.multiple_of(wid * EPT, 8)

        zv = jnp.zeros((16,), jnp.float32)

        def zrow(i, carry):
            for j in range(D // 16):
                rows0[i, pl.ds(j * 16, 16)] = zv
            return carry

        lax.fori_loop(0, CH, zrow, 0)
        for k in range(ZFULL):
            pltpu.sync_copy(rows0, acc_sh.at[pl.ds(r0 + k * CH, CH)])
        pltpu.sync_copy(rows0.at[pl.ds(0, ZREM)],
                        acc_sh.at[pl.ds(r0 + ZFULL * CH, ZREM)])
        if with_deg:
            for j in range(CH // 16):
                ones_v[pl.ds(j * 16, 16)] = jnp.ones((16,), jnp.float32)
            for j in range(RPT // 16):
                degrow_v[pl.ds(j * 16, 16)] = zv
            pltpu.sync_copy(degrow_v, deg_sh.at[pl.ds(r0, RPT)])

        @pl.when(s == NS - 1)
        def _zero_tail():
            pltpu.sync_copy(rows0.at[pl.ds(0, R_TAIL)],
                            acc_sh.at[pl.ds(R_TAIL0, R_TAIL)])
            if with_deg:
                pltpu.sync_copy(degrow_v.at[pl.ds(0, R_TAIL)],
                                deg_sh.at[pl.ds(R_TAIL0, R_TAIL)])

        plsc.subcore_barrier()

        pltpu.sync_copy(src.at[pl.ds(ebase, EPT)], src_all)

        def src_sl(i):
            return src_all.at[pl.ds(pl.multiple_of(i * CH, 8), CH)]

        def start_chunk(i, q):
            dst_v, rows, isem, gsem = sets[q][0], sets[q][1], sets[q][2], sets[q][3]
            pltpu.async_copy(dst.at[pl.ds(pl.multiple_of(ebase + i * CH, 8), CH)],
                             dst_v, isem)
            pltpu.async_copy(feat.at[src_sl(i)], rows, gsem)

        def start_scatter(q):
            dst_v, rows, ssem, dsem = sets[q][0], sets[q][1], sets[q][4], sets[q][5]
            pltpu.async_copy(rows, acc_sh.at[dst_v], ssem, add=True)
            if with_deg:
                pltpu.async_copy(ones_v, deg_sh.at[dst_v], dsem, add=True)

        def wait_idx(q):
            dst_v, isem = sets[q][0], sets[q][2]
            pltpu.make_async_copy(dst.at[pl.ds(ebase, CH)], dst_v, isem).wait()

        def wait_gather(q):
            rows, gsem = sets[q][1], sets[q][3]
            pltpu.make_async_copy(feat.at[src_sl(0)], rows, gsem).wait()

        def wait_scatter(q):
            dst_v, rows, ssem, dsem = sets[q][0], sets[q][1], sets[q][4], sets[q][5]
            pltpu.make_async_copy(rows, acc_sh.at[dst_v], ssem).wait()
            if with_deg:
                pltpu.make_async_copy(ones_v, deg_sh.at[dst_v], dsem).wait()

        rem_lo = pl.multiple_of(NFULL * CH, 8)
        pltpu.sync_copy(dst.at[pl.ds(ebase + rem_lo, REM_E)], dst_rem)
        pltpu.async_copy(feat.at[src_all.at[pl.ds(rem_lo, REM_E)]],
                         rows0.at[pl.ds(0, REM_E)], sets[0][3]).wait()
        d_rem = pltpu.async_copy(rows0.at[pl.ds(0, REM_E)],
                                 acc_sh.at[dst_rem], sets[0][4], add=True)
        if with_deg:
            pltpu.async_copy(ones_v.at[pl.ds(0, REM_E)],
                             deg_sh.at[dst_rem], sets[0][5], add=True).wait()
        d_rem.wait()
        for t, i in enumerate(range(NMAIN, NFULL)):
            start_chunk(i, t)
        for t, i in enumerate(range(NMAIN, NFULL)):
            wait_gather(t)
            wait_idx(t)
            start_scatter(t)
        for t, i in enumerate(range(NMAIN, NFULL)):
            wait_scatter(t)

        for q in range(NSETS):
            start_chunk(q, q)

        def quad(g, carry):
            i = NSETS * g
            for q in range(NSETS):
                wait_gather(q)
                wait_idx(q)
                start_scatter(q)
            for q in range(NSETS):
                nq = jnp.where(i + q + NSETS < NMAIN, i + q + NSETS, 0)
                wait_scatter(q)
                start_chunk(nq, q)
            return carry

        lax.fori_loop(0, NQUAD, quad, 0)
        for q in range(NSETS):
            wait_idx(q)
            wait_gather(q)
        plsc.subcore_barrier()

        pltpu.sync_copy(acc_sh.at[pl.ds(r0, RPT)], out.at[c, pl.ds(r0, RPT)])
        if with_deg:
            pltpu.sync_copy(deg_sh.at[pl.ds(r0, RPT)], degrow_v)
            d0 = pl.multiple_of(c * N + r0, 8)
            pltpu.sync_copy(degrow_v, deg_out.at[pl.ds(d0, RPT)])

        @pl.when(s == NS - 1)
        def _write_tail():
            pltpu.sync_copy(acc_sh.at[pl.ds(R_TAIL0, R_TAIL)],
                            out.at[c, pl.ds(R_TAIL0, R_TAIL)])
            if with_deg:
                pltpu.sync_copy(deg_sh.at[pl.ds(R_TAIL0, R_TAIL)],
                                degrow_v.at[pl.ds(0, R_TAIL)])
                dt = pl.multiple_of(c * N + R_TAIL0, 8)
                pltpu.sync_copy(degrow_v.at[pl.ds(0, R_TAIL)],
                                deg_out.at[pl.ds(dt, R_TAIL)])

    return pl.kernel(
        body,
        out_type=out_type,
        mesh=plsc.VectorSubcoreMesh(core_axis_name="c", subcore_axis_name="s"),
        scratch_types=scratch,
        name=f"sc_agg_d{D}" + ("_deg" if with_deg else ""),
    )


_sc_agg_deg = _make_sc_agg(F_HID, with_deg=True)
_sc_agg = _make_sc_agg(F_HID, with_deg=False)


_BR = 1000


def _mm2_body(x_ref, wn_ref, ws_ref, b_ref, on_ref, os_ref):
    xb = x_ref[...]
    on_ref[...] = jnp.dot(xb, wn_ref[...], preferred_element_type=jnp.float32)
    os_ref[...] = jnp.dot(xb, ws_ref[...], preferred_element_type=jnp.float32) + b_ref[...]


def _tc_mm2(x, W_neigh, W_self, b):
    return pl.pallas_call(
        _mm2_body,
        grid=(N // _BR,),
        in_specs=[
            pl.BlockSpec((_BR, F_IN), lambda i: (i, 0)),
            pl.BlockSpec((F_IN, F_HID), lambda i: (0, 0)),
            pl.BlockSpec((F_IN, F_HID), lambda i: (0, 0)),
            pl.BlockSpec((1, F_HID), lambda i: (0, 0)),
        ],
        out_specs=[
            pl.BlockSpec((_BR, F_HID), lambda i: (i, 0)),
            pl.BlockSpec((_BR, F_HID), lambda i: (i, 0)),
        ],
        out_shape=[
            jax.ShapeDtypeStruct((N, F_HID), jnp.float32),
            jax.ShapeDtypeStruct((N, F_HID), jnp.float32),
        ],
    )(x, W_neigh, W_self, b.reshape(1, -1))


def _mm_body(x_ref, w_ref, b_ref, o_ref):
    o_ref[...] = jnp.dot(x_ref[...], w_ref[...],
                         preferred_element_type=jnp.float32) + b_ref[...]


def _tc_mm(x, W, b):
    Din, Dout = W.shape
    return pl.pallas_call(
        _mm_body,
        grid=(N // _BR,),
        in_specs=[
            pl.BlockSpec((_BR, Din), lambda i: (i, 0)),
            pl.BlockSpec((Din, Dout), lambda i: (0, 0)),
            pl.BlockSpec((1, Dout), lambda i: (0, 0)),
        ],
        out_specs=pl.BlockSpec((_BR, Dout), lambda i: (i, 0)),
        out_shape=jax.ShapeDtypeStruct((N, Dout), jnp.float32),
    )(x, W, b.reshape(1, -1))


def _relu_comb_body(xs_ref, agg_ref, deg_ref, h_ref):
    deg = jnp.maximum(deg_ref[0] + deg_ref[1], 1.0)
    h_ref[...] = jnp.maximum(xs_ref[...] + (agg_ref[0] + agg_ref[1]) / deg, 0.0)


def _tc_relu_comb(xs, agg, deg2):
    return pl.pallas_call(
        _relu_comb_body,
        grid=(N // _BR,),
        in_specs=[
            pl.BlockSpec((_BR, F_HID), lambda i: (i, 0)),
            pl.BlockSpec((NC, _BR, F_HID), lambda i: (0, i, 0)),
            pl.BlockSpec((NC, _BR, 1), lambda i: (0, i, 0)),
        ],
        out_specs=pl.BlockSpec((_BR, F_HID), lambda i: (i, 0)),
        out_shape=jax.ShapeDtypeStruct((N, F_HID), jnp.float32),
    )(xs, agg, deg2)


def _comb_body(hs_ref, agg_ref, deg_ref, wn_ref, out_ref):
    deg = jnp.maximum(deg_ref[0] + deg_ref[1], 1.0)
    h_neigh = (agg_ref[0] + agg_ref[1]) / deg
    out_ref[...] = hs_ref[...] + jnp.dot(
        h_neigh, wn_ref[...], preferred_element_type=jnp.float32)


def _tc_combine(hs, agg, deg2, W_neigh):
    return pl.pallas_call(
        _comb_body,
        grid=(N // _BR,),
        in_specs=[
            pl.BlockSpec((_BR, F_OUT), lambda i: (i, 0)),
            pl.BlockSpec((NC, _BR, F_HID), lambda i: (0, i, 0)),
            pl.BlockSpec((NC, _BR, 1), lambda i: (0, i, 0)),
            pl.BlockSpec((F_HID, F_OUT), lambda i: (0, 0)),
        ],
        out_specs=pl.BlockSpec((_BR, F_OUT), lambda i: (i, 0)),
        out_shape=jax.ShapeDtypeStruct((N, F_OUT), jnp.float32),
    )(hs, agg, deg2, W_neigh)


def kernel(x, edge_index, W_self1, W_neigh1, b1, W_self2, W_neigh2, b2):
    src = edge_index[0].astype(jnp.int32)
    dst = edge_index[1].astype(jnp.int32)

    zb1 = jnp.zeros((F_HID,), jnp.float32)
    xn1 = _tc_mm(x, W_neigh1, zb1)
    agg1, deg = _sc_agg_deg(xn1, src, dst)
    xs1 = _tc_mm(x, W_self1, b1)
    deg2 = deg.reshape(NC, N, 1)
    h = _tc_relu_comb(xs1, agg1, deg2)
    agg2 = _sc_agg(h, src, dst)
    hs2 = _tc_mm(h, W_self2, b2)
    return _tc_combine(hs2, agg2, deg2, W_neigh2)


# speedup vs baseline: 1.0350x; 1.0350x over previous
import jax
import jax.numpy as jnp
from jax import lax
from jax.experimental import pallas as pl
from jax.experimental.pallas import tpu as pltpu
from jax.experimental.pallas import tpu_sc as plsc

N = 10000
E = 320000
F_IN = 128
F_HID = 128
F_OUT = 64

NC = 2
NS = 16
NW = NC * NS
EPT = E // NW
CH = 64
NFULL = EPT // CH
REM_E = EPT - NFULL * CH
NSETS = 4
NQUAD = NFULL // NSETS
NMAIN = NQUAD * NSETS
RPT = 624
R_TAIL0 = NS * RPT
R_TAIL = N - R_TAIL0
ZFULL = RPT // CH
ZREM = RPT - ZFULL * CH


def _make_sc_agg(D, with_deg):
    out_type = jax.ShapeDtypeStruct((NC, N, D), jnp.float32)
    if with_deg:
        out_type = (out_type, jax.ShapeDtypeStruct((NC * N,), jnp.float32))
    scratch = [
        pltpu.VMEM_SHARED((N, D), jnp.float32),
        pltpu.VMEM((EPT,), jnp.int32),
        pltpu.VMEM((REM_E,), jnp.int32),
    ]
    for _ in range(NSETS):
        scratch += [
            pltpu.VMEM((CH,), jnp.int32),
            pltpu.VMEM((CH, D), jnp.float32),
            pltpu.SemaphoreType.DMA,
            pltpu.SemaphoreType.DMA,
            pltpu.SemaphoreType.DMA,
            pltpu.SemaphoreType.DMA,
        ]
    if with_deg:
        scratch += [
            pltpu.VMEM_SHARED((N,), jnp.float32),
            pltpu.VMEM((CH,), jnp.float32),
            pltpu.VMEM((RPT,), jnp.float32),
        ]

    def body(feat, src, dst, *refs):
        if with_deg:
            out, deg_out = refs[0], refs[1]
            rest = refs[2:]
        else:
            out = refs[0]
            rest = refs[1:]
        acc_sh, src_all, dst_rem = rest[0], rest[1], rest[2]
        sets = [tuple(rest[3 + 6 * q: 3 + 6 * (q + 1)]) for q in range(NSETS)]
        if with_deg:
            deg_sh, ones_v, degrow_v = rest[3 + 6 * NSETS:]
        rows0 = sets[0][1]
        c = lax.axis_index("c")
        s = lax.axis_index("s")
        wid = c * NS + s
        r0 = pl.multiple_of(s * RPT, 8)
        ebase = pl.multiple_of(wid * EPT, 8)

        zv = jnp.zeros((16,), jnp.float32)

        def zrow(i, carry):
            for j in range(D // 16):
                rows0[i, pl.ds(j * 16, 16)] = zv
            return carry

        lax.fori_loop(0, CH, zrow, 0)
        for k in range(ZFULL):
            pltpu.sync_copy(rows0, acc_sh.at[pl.ds(r0 + k * CH, CH)])
        pltpu.sync_copy(rows0.at[pl.ds(0, ZREM)],
                        acc_sh.at[pl.ds(r0 + ZFULL * CH, ZREM)])
        if with_deg:
            for j in range(CH // 16):
                ones_v[pl.ds(j * 16, 16)] = jnp.ones((16,), jnp.float32)
            for j in range(RPT // 16):
                degrow_v[pl.ds(j * 16, 16)] = zv
            pltpu.sync_copy(degrow_v, deg_sh.at[pl.ds(r0, RPT)])

        @pl.when(s == NS - 1)
        def _zero_tail():
            pltpu.sync_copy(rows0.at[pl.ds(0, R_TAIL)],
                            acc_sh.at[pl.ds(R_TAIL0, R_TAIL)])
            if with_deg:
                pltpu.sync_copy(degrow_v.at[pl.ds(0, R_TAIL)],
                                deg_sh.at[pl.ds(R_TAIL0, R_TAIL)])

        plsc.subcore_barrier()

        pltpu.sync_copy(src.at[pl.ds(ebase, EPT)], src_all)

        def src_sl(i):
            return src_all.at[pl.ds(pl.multiple_of(i * CH, 8), CH)]

        def start_chunk(i, q):
            dst_v, rows, isem, gsem = sets[q][0], sets[q][1], sets[q][2], sets[q][3]
            pltpu.async_copy(dst.at[pl.ds(pl.multiple_of(ebase + i * CH, 8), CH)],
                             dst_v, isem)
            pltpu.async_copy(feat.at[src_sl(i)], rows, gsem)

        def start_scatter(q):
            dst_v, rows, ssem, dsem = sets[q][0], sets[q][1], sets[q][4], sets[q][5]
            pltpu.async_copy(rows, acc_sh.at[dst_v], ssem, add=True)
            if with_deg:
                pltpu.async_copy(ones_v, deg_sh.at[dst_v], dsem, add=True)

        def wait_idx(q):
            dst_v, isem = sets[q][0], sets[q][2]
            pltpu.make_async_copy(dst.at[pl.ds(ebase, CH)], dst_v, isem).wait()

        def wait_gather(q):
            rows, gsem = sets[q][1], sets[q][3]
            pltpu.make_async_copy(feat.at[src_sl(0)], rows, gsem).wait()

        def wait_scatter(q):
            dst_v, rows, ssem, dsem = sets[q][0], sets[q][1], sets[q][4], sets[q][5]
            pltpu.make_async_copy(rows, acc_sh.at[dst_v], ssem).wait()
            if with_deg:
                pltpu.make_async_copy(ones_v, deg_sh.at[dst_v], dsem).wait()

        rem_lo = pl.multiple_of(NFULL * CH, 8)
        pltpu.sync_copy(dst.at[pl.ds(ebase + rem_lo, REM_E)], dst_rem)
        pltpu.async_copy(feat.at[src_all.at[pl.ds(rem_lo, REM_E)]],
                         rows0.at[pl.ds(0, REM_E)], sets[0][3]).wait()
        d_rem = pltpu.async_copy(rows0.at[pl.ds(0, REM_E)],
                                 acc_sh.at[dst_rem], sets[0][4], add=True)
        if with_deg:
            pltpu.async_copy(ones_v.at[pl.ds(0, REM_E)],
                             deg_sh.at[dst_rem], sets[0][5], add=True).wait()
        d_rem.wait()
        for t, i in enumerate(range(NMAIN, NFULL)):
            start_chunk(i, t)
        for t, i in enumerate(range(NMAIN, NFULL)):
            wait_gather(t)
            wait_idx(t)
            start_scatter(t)
        for t, i in enumerate(range(NMAIN, NFULL)):
            wait_scatter(t)

        for q in range(NSETS):
            start_chunk(q, q)

        def quad(g, carry):
            i = NSETS * g
            for q in range(NSETS):
                wait_gather(q)
                wait_idx(q)
                start_scatter(q)
            for q in range(NSETS):
                nq = jnp.where(i + q + NSETS < NMAIN, i + q + NSETS, 0)
                wait_scatter(q)
                start_chunk(nq, q)
            return carry

        lax.fori_loop(0, NQUAD, quad, 0)
        for q in range(NSETS):
            wait_idx(q)
            wait_gather(q)
        plsc.subcore_barrier()

        def wr_start(lo, nrows, q):
            rows, ssem = sets[q][1], sets[q][4]
            pltpu.sync_copy(acc_sh.at[pl.ds(lo, nrows)],
                            rows.at[pl.ds(0, nrows)])
            pltpu.async_copy(rows.at[pl.ds(0, nrows)],
                             out.at[c, pl.ds(lo, nrows)], ssem)

        def wr_wait(lo, nrows, q):
            rows, ssem = sets[q][1], sets[q][4]
            pltpu.make_async_copy(rows.at[pl.ds(0, nrows)],
                                  out.at[c, pl.ds(lo, nrows)], ssem).wait()

        lo_rem = pl.multiple_of(r0 + ZFULL * CH, 8)
        outstanding = {}
        for k in range(ZFULL):
            q = k % NSETS
            if q in outstanding:
                wr_wait(lo_rem, outstanding.pop(q), q)
            wr_start(pl.multiple_of(r0 + k * CH, 8), CH, q)
            outstanding[q] = CH
        q = ZFULL % NSETS
        if q in outstanding:
            wr_wait(lo_rem, outstanding.pop(q), q)
        wr_start(lo_rem, ZREM, q)
        outstanding[q] = ZREM
        for q2, nr in outstanding.items():
            wr_wait(lo_rem, nr, q2)
        if with_deg:
            pltpu.sync_copy(deg_sh.at[pl.ds(r0, RPT)], degrow_v)
            d0 = pl.multiple_of(c * N + r0, 8)
            pltpu.sync_copy(degrow_v, deg_out.at[pl.ds(d0, RPT)])

        @pl.when(s == NS - 1)
        def _write_tail():
            pltpu.sync_copy(acc_sh.at[pl.ds(R_TAIL0, R_TAIL)],
                            rows0.at[pl.ds(0, R_TAIL)])
            pltpu.sync_copy(rows0.at[pl.ds(0, R_TAIL)],
                            out.at[c, pl.ds(R_TAIL0, R_TAIL)])
            if with_deg:
                pltpu.sync_copy(deg_sh.at[pl.ds(R_TAIL0, R_TAIL)],
                                degrow_v.at[pl.ds(0, R_TAIL)])
                dt = pl.multiple_of(c * N + R_TAIL0, 8)
                pltpu.sync_copy(degrow_v.at[pl.ds(0, R_TAIL)],
                                deg_out.at[pl.ds(dt, R_TAIL)])

    return pl.kernel(
        body,
        out_type=out_type,
        mesh=plsc.VectorSubcoreMesh(core_axis_name="c", subcore_axis_name="s"),
        scratch_types=scratch,
        name=f"sc_agg_d{D}" + ("_deg" if with_deg else ""),
    )


_sc_agg_deg = _make_sc_agg(F_HID, with_deg=True)
_sc_agg = _make_sc_agg(F_HID, with_deg=False)


_BR = 2000


def _mm2_body(x_ref, wn_ref, ws_ref, b_ref, on_ref, os_ref):
    xb = x_ref[...]
    on_ref[...] = jnp.dot(xb, wn_ref[...], preferred_element_type=jnp.float32)
    os_ref[...] = jnp.dot(xb, ws_ref[...], preferred_element_type=jnp.float32) + b_ref[...]


def _tc_mm2(x, W_neigh, W_self, b):
    return pl.pallas_call(
        _mm2_body,
        grid=(N // _BR,),
        in_specs=[
            pl.BlockSpec((_BR, F_IN), lambda i: (i, 0)),
            pl.BlockSpec((F_IN, F_HID), lambda i: (0, 0)),
            pl.BlockSpec((F_IN, F_HID), lambda i: (0, 0)),
            pl.BlockSpec((1, F_HID), lambda i: (0, 0)),
        ],
        out_specs=[
            pl.BlockSpec((_BR, F_HID), lambda i: (i, 0)),
            pl.BlockSpec((_BR, F_HID), lambda i: (i, 0)),
        ],
        out_shape=[
            jax.ShapeDtypeStruct((N, F_HID), jnp.float32),
            jax.ShapeDtypeStruct((N, F_HID), jnp.float32),
        ],
    )(x, W_neigh, W_self, b.reshape(1, -1))


def _mm_body(x_ref, w_ref, b_ref, o_ref):
    o_ref[...] = jnp.dot(x_ref[...], w_ref[...],
                         preferred_element_type=jnp.float32) + b_ref[...]


def _tc_mm(x, W, b):
    Din, Dout = W.shape
    return pl.pallas_call(
        _mm_body,
        grid=(N // _BR,),
        in_specs=[
            pl.BlockSpec((_BR, Din), lambda i: (i, 0)),
            pl.BlockSpec((Din, Dout), lambda i: (0, 0)),
            pl.BlockSpec((1, Dout), lambda i: (0, 0)),
        ],
        out_specs=pl.BlockSpec((_BR, Dout), lambda i: (i, 0)),
        out_shape=jax.ShapeDtypeStruct((N, Dout), jnp.float32),
    )(x, W, b.reshape(1, -1))


def _relu_comb_body(xs_ref, agg_ref, deg_ref, h_ref):
    deg = jnp.maximum(deg_ref[0] + deg_ref[1], 1.0)
    h_ref[...] = jnp.maximum(xs_ref[...] + (agg_ref[0] + agg_ref[1]) / deg, 0.0)


def _tc_relu_comb(xs, agg, deg2):
    return pl.pallas_call(
        _relu_comb_body,
        grid=(N // _BR,),
        in_specs=[
            pl.BlockSpec((_BR, F_HID), lambda i: (i, 0)),
            pl.BlockSpec((NC, _BR, F_HID), lambda i: (0, i, 0)),
            pl.BlockSpec((NC, _BR, 1), lambda i: (0, i, 0)),
        ],
        out_specs=pl.BlockSpec((_BR, F_HID), lambda i: (i, 0)),
        out_shape=jax.ShapeDtypeStruct((N, F_HID), jnp.float32),
    )(xs, agg, deg2)


def _comb_body(hs_ref, agg_ref, deg_ref, wn_ref, out_ref):
    deg = jnp.maximum(deg_ref[0] + deg_ref[1], 1.0)
    h_neigh = (agg_ref[0] + agg_ref[1]) / deg
    out_ref[...] = hs_ref[...] + jnp.dot(
        h_neigh, wn_ref[...], preferred_element_type=jnp.float32)


def _tc_combine(hs, agg, deg2, W_neigh):
    return pl.pallas_call(
        _comb_body,
        grid=(N // _BR,),
        in_specs=[
            pl.BlockSpec((_BR, F_OUT), lambda i: (i, 0)),
            pl.BlockSpec((NC, _BR, F_HID), lambda i: (0, i, 0)),
            pl.BlockSpec((NC, _BR, 1), lambda i: (0, i, 0)),
            pl.BlockSpec((F_HID, F_OUT), lambda i: (0, 0)),
        ],
        out_specs=pl.BlockSpec((_BR, F_OUT), lambda i: (i, 0)),
        out_shape=jax.ShapeDtypeStruct((N, F_OUT), jnp.float32),
    )(hs, agg, deg2, W_neigh)


def kernel(x, edge_index, W_self1, W_neigh1, b1, W_self2, W_neigh2, b2):
    src = edge_index[0].astype(jnp.int32)
    dst = edge_index[1].astype(jnp.int32)

    zb1 = jnp.zeros((F_HID,), jnp.float32)
    xn1 = _tc_mm(x, W_neigh1, zb1)
    agg1, deg = _sc_agg_deg(xn1, src, dst)
    xs1 = _tc_mm(x, W_self1, b1)
    deg2 = deg.reshape(NC, N, 1)
    h = _tc_relu_comb(xs1, agg1, deg2)
    agg2 = _sc_agg(h, src, dst)
    hs2 = _tc_mm(h, W_self2, b2)
    return _tc_combine(hs2, agg2, deg2, W_neigh2)


# speedup vs baseline: 1.0513x; 1.0157x over previous
import jax
import jax.numpy as jnp
from jax import lax
from jax.experimental import pallas as pl
from jax.experimental.pallas import tpu as pltpu
from jax.experimental.pallas import tpu_sc as plsc

N = 10000
E = 320000
F_IN = 128
F_HID = 128
F_OUT = 64

NC = 2
NS = 16
NW = NC * NS
EPT = E // NW
CH = 64
NFULL = EPT // CH
REM_E = EPT - NFULL * CH
NSETS = 4
NQUAD = NFULL // NSETS
NMAIN = NQUAD * NSETS
RPT = 624
R_TAIL0 = NS * RPT
R_TAIL = N - R_TAIL0
ZFULL = RPT // CH
ZREM = RPT - ZFULL * CH


def _make_sc_agg(D, with_deg):
    out_type = jax.ShapeDtypeStruct((NC, N, D), jnp.float32)
    if with_deg:
        out_type = (out_type, jax.ShapeDtypeStruct((NC * N,), jnp.float32))
    scratch = [
        pltpu.VMEM_SHARED((N, D), jnp.float32),
        pltpu.VMEM((EPT,), jnp.int32),
        pltpu.VMEM((REM_E,), jnp.int32),
        pltpu.VMEM((REM_E, D), jnp.float32),
        pltpu.SemaphoreType.DMA,
    ]
    for _ in range(NSETS):
        scratch += [
            pltpu.VMEM((CH,), jnp.int32),
            pltpu.VMEM((CH, D), jnp.float32),
            pltpu.SemaphoreType.DMA,
            pltpu.SemaphoreType.DMA,
            pltpu.SemaphoreType.DMA,
            pltpu.SemaphoreType.DMA,
        ]
    if with_deg:
        scratch += [
            pltpu.VMEM_SHARED((N,), jnp.float32),
            pltpu.VMEM((CH,), jnp.float32),
            pltpu.VMEM((RPT,), jnp.float32),
        ]

    def body(feat, src, dst, *refs):
        if with_deg:
            out, deg_out = refs[0], refs[1]
            rest = refs[2:]
        else:
            out = refs[0]
            rest = refs[1:]
        acc_sh, src_all, dst_rem, rows_rem, rsem = rest[:5]
        sets = [tuple(rest[5 + 6 * q: 5 + 6 * (q + 1)]) for q in range(NSETS)]
        if with_deg:
            deg_sh, ones_v, degrow_v = rest[5 + 6 * NSETS:]
        rows0 = sets[0][1]
        c = lax.axis_index("c")
        s = lax.axis_index("s")
        wid = c * NS + s
        r0 = pl.multiple_of(s * RPT, 8)
        ebase = pl.multiple_of(wid * EPT, 8)

        zv = jnp.zeros((16,), jnp.float32)

        def zrow(i, carry):
            for j in range(D // 16):
                rows0[i, pl.ds(j * 16, 16)] = zv
            return carry

        lax.fori_loop(0, CH, zrow, 0)
        zdescs = []
        for k in range(ZFULL):
            zdescs.append(pltpu.async_copy(
                rows0, acc_sh.at[pl.ds(r0 + k * CH, CH)], sets[k % NSETS][4]))
        zdescs.append(pltpu.async_copy(
            rows0.at[pl.ds(0, ZREM)],
            acc_sh.at[pl.ds(r0 + ZFULL * CH, ZREM)], sets[ZFULL % NSETS][4]))
        if with_deg:
            for j in range(CH // 16):
                ones_v[pl.ds(j * 16, 16)] = jnp.ones((16,), jnp.float32)
            for j in range(RPT // 16):
                degrow_v[pl.ds(j * 16, 16)] = zv
            zdescs.append(pltpu.async_copy(
                degrow_v, deg_sh.at[pl.ds(r0, RPT)], sets[0][5]))
        pltpu.sync_copy(src.at[pl.ds(ebase, EPT)], src_all)
        for d in zdescs:
            d.wait()

        @pl.when(s == NS - 1)
        def _zero_tail():
            pltpu.sync_copy(rows0.at[pl.ds(0, R_TAIL)],
                            acc_sh.at[pl.ds(R_TAIL0, R_TAIL)])
            if with_deg:
                pltpu.sync_copy(degrow_v.at[pl.ds(0, R_TAIL)],
                                deg_sh.at[pl.ds(R_TAIL0, R_TAIL)])

        plsc.subcore_barrier()

        def src_sl(i):
            return src_all.at[pl.ds(pl.multiple_of(i * CH, 8), CH)]

        def start_chunk(i, q):
            dst_v, rows, isem, gsem = sets[q][0], sets[q][1], sets[q][2], sets[q][3]
            pltpu.async_copy(dst.at[pl.ds(pl.multiple_of(ebase + i * CH, 8), CH)],
                             dst_v, isem)
            pltpu.async_copy(feat.at[src_sl(i)], rows, gsem)

        def start_scatter(q):
            dst_v, rows, ssem, dsem = sets[q][0], sets[q][1], sets[q][4], sets[q][5]
            pltpu.async_copy(rows, acc_sh.at[dst_v], ssem, add=True)
            if with_deg:
                pltpu.async_copy(ones_v, deg_sh.at[dst_v], dsem, add=True)

        def wait_idx(q):
            dst_v, isem = sets[q][0], sets[q][2]
            pltpu.make_async_copy(dst.at[pl.ds(ebase, CH)], dst_v, isem).wait()

        def wait_gather(q):
            rows, gsem = sets[q][1], sets[q][3]
            pltpu.make_async_copy(feat.at[src_sl(0)], rows, gsem).wait()

        def wait_scatter(q):
            dst_v, rows, ssem, dsem = sets[q][0], sets[q][1], sets[q][4], sets[q][5]
            pltpu.make_async_copy(rows, acc_sh.at[dst_v], ssem).wait()
            if with_deg:
                pltpu.make_async_copy(ones_v, deg_sh.at[dst_v], dsem).wait()

        rem_lo = pl.multiple_of(NFULL * CH, 8)
        pltpu.sync_copy(dst.at[pl.ds(ebase + rem_lo, REM_E)], dst_rem)
        g_rem = pltpu.async_copy(feat.at[src_all.at[pl.ds(rem_lo, REM_E)]],
                                 rows_rem, rsem)
        for t, i in enumerate(range(NMAIN, NFULL)):
            start_chunk(i, t)
        for t, i in enumerate(range(NMAIN, NFULL)):
            wait_gather(t)
            wait_idx(t)
            start_scatter(t)
        for t, i in enumerate(range(NMAIN, NFULL)):
            wait_scatter(t)

        for q in range(NSETS):
            start_chunk(q, q)

        def quad(g, carry):
            i = NSETS * g
            for q in range(NSETS):
                wait_gather(q)
                wait_idx(q)
                start_scatter(q)
            for q in range(NSETS):
                nq = jnp.where(i + q + NSETS < NMAIN, i + q + NSETS, 0)
                wait_scatter(q)
                start_chunk(nq, q)
            return carry

        lax.fori_loop(0, NQUAD, quad, 0)
        for q in range(NSETS):
            wait_idx(q)
            wait_gather(q)
        g_rem.wait()
        d_rem = pltpu.async_copy(rows_rem, acc_sh.at[dst_rem],
                                 sets[0][4], add=True)
        if with_deg:
            pltpu.async_copy(ones_v.at[pl.ds(0, REM_E)],
                             deg_sh.at[dst_rem], sets[0][5], add=True).wait()
        d_rem.wait()
        plsc.subcore_barrier()

        def wr_start(lo, nrows, q):
            rows, ssem = sets[q][1], sets[q][4]
            pltpu.sync_copy(acc_sh.at[pl.ds(lo, nrows)],
                            rows.at[pl.ds(0, nrows)])
            pltpu.async_copy(rows.at[pl.ds(0, nrows)],
                             out.at[c, pl.ds(lo, nrows)], ssem)

        def wr_wait(lo, nrows, q):
            rows, ssem = sets[q][1], sets[q][4]
            pltpu.make_async_copy(rows.at[pl.ds(0, nrows)],
                                  out.at[c, pl.ds(lo, nrows)], ssem).wait()

        lo_rem = pl.multiple_of(r0 + ZFULL * CH, 8)
        outstanding = {}
        for k in range(ZFULL):
            q = k % NSETS
            if q in outstanding:
                wr_wait(lo_rem, outstanding.pop(q), q)
            wr_start(pl.multiple_of(r0 + k * CH, 8), CH, q)
            outstanding[q] = CH
        q = ZFULL % NSETS
        if q in outstanding:
            wr_wait(lo_rem, outstanding.pop(q), q)
        wr_start(lo_rem, ZREM, q)
        outstanding[q] = ZREM
        for q2, nr in outstanding.items():
            wr_wait(lo_rem, nr, q2)
        if with_deg:
            pltpu.sync_copy(deg_sh.at[pl.ds(r0, RPT)], degrow_v)
            d0 = pl.multiple_of(c * N + r0, 8)
            pltpu.sync_copy(degrow_v, deg_out.at[pl.ds(d0, RPT)])

        @pl.when(s == NS - 1)
        def _write_tail():
            pltpu.sync_copy(acc_sh.at[pl.ds(R_TAIL0, R_TAIL)],
                            rows0.at[pl.ds(0, R_TAIL)])
            pltpu.sync_copy(rows0.at[pl.ds(0, R_TAIL)],
                            out.at[c, pl.ds(R_TAIL0, R_TAIL)])
            if with_deg:
                pltpu.sync_copy(deg_sh.at[pl.ds(R_TAIL0, R_TAIL)],
                                degrow_v.at[pl.ds(0, R_TAIL)])
                dt = pl.multiple_of(c * N + R_TAIL0, 8)
                pltpu.sync_copy(degrow_v.at[pl.ds(0, R_TAIL)],
                                deg_out.at[pl.ds(dt, R_TAIL)])

    return pl.kernel(
        body,
        out_type=out_type,
        mesh=plsc.VectorSubcoreMesh(core_axis_name="c", subcore_axis_name="s"),
        scratch_types=scratch,
        name=f"sc_agg_d{D}" + ("_deg" if with_deg else ""),
    )


_sc_agg_deg = _make_sc_agg(F_HID, with_deg=True)
_sc_agg = _make_sc_agg(F_HID, with_deg=False)


_BR = 2000


def _mm2_body(x_ref, wn_ref, ws_ref, b_ref, on_ref, os_ref):
    xb = x_ref[...]
    on_ref[...] = jnp.dot(xb, wn_ref[...], preferred_element_type=jnp.float32)
    os_ref[...] = jnp.dot(xb, ws_ref[...], preferred_element_type=jnp.float32) + b_ref[...]


def _tc_mm2(x, W_neigh, W_self, b):
    return pl.pallas_call(
        _mm2_body,
        grid=(N // _BR,),
        in_specs=[
            pl.BlockSpec((_BR, F_IN), lambda i: (i, 0)),
            pl.BlockSpec((F_IN, F_HID), lambda i: (0, 0)),
            pl.BlockSpec((F_IN, F_HID), lambda i: (0, 0)),
            pl.BlockSpec((1, F_HID), lambda i: (0, 0)),
        ],
        out_specs=[
            pl.BlockSpec((_BR, F_HID), lambda i: (i, 0)),
            pl.BlockSpec((_BR, F_HID), lambda i: (i, 0)),
        ],
        out_shape=[
            jax.ShapeDtypeStruct((N, F_HID), jnp.float32),
            jax.ShapeDtypeStruct((N, F_HID), jnp.float32),
        ],
    )(x, W_neigh, W_self, b.reshape(1, -1))


def _mm_body(x_ref, w_ref, b_ref, o_ref):
    o_ref[...] = jnp.dot(x_ref[...], w_ref[...],
                         preferred_element_type=jnp.float32) + b_ref[...]


def _tc_mm(x, W, b):
    Din, Dout = W.shape
    return pl.pallas_call(
        _mm_body,
        grid=(N // _BR,),
        in_specs=[
            pl.BlockSpec((_BR, Din), lambda i: (i, 0)),
            pl.BlockSpec((Din, Dout), lambda i: (0, 0)),
            pl.BlockSpec((1, Dout), lambda i: (0, 0)),
        ],
        out_specs=pl.BlockSpec((_BR, Dout), lambda i: (i, 0)),
        out_shape=jax.ShapeDtypeStruct((N, Dout), jnp.float32),
    )(x, W, b.reshape(1, -1))


def _relu_comb_body(xs_ref, agg_ref, deg_ref, h_ref):
    deg = jnp.maximum(deg_ref[0] + deg_ref[1], 1.0)
    h_ref[...] = jnp.maximum(xs_ref[...] + (agg_ref[0] + agg_ref[1]) / deg, 0.0)


def _tc_relu_comb(xs, agg, deg2):
    return pl.pallas_call(
        _relu_comb_body,
        grid=(N // _BR,),
        in_specs=[
            pl.BlockSpec((_BR, F_HID), lambda i: (i, 0)),
            pl.BlockSpec((NC, _BR, F_HID), lambda i: (0, i, 0)),
            pl.BlockSpec((NC, _BR, 1), lambda i: (0, i, 0)),
        ],
        out_specs=pl.BlockSpec((_BR, F_HID), lambda i: (i, 0)),
        out_shape=jax.ShapeDtypeStruct((N, F_HID), jnp.float32),
    )(xs, agg, deg2)


def _comb_body(hs_ref, agg_ref, deg_ref, wn_ref, out_ref):
    deg = jnp.maximum(deg_ref[0] + deg_ref[1], 1.0)
    h_neigh = (agg_ref[0] + agg_ref[1]) / deg
    out_ref[...] = hs_ref[...] + jnp.dot(
        h_neigh, wn_ref[...], preferred_element_type=jnp.float32)


def _tc_combine(hs, agg, deg2, W_neigh):
    return pl.pallas_call(
        _comb_body,
        grid=(N // _BR,),
        in_specs=[
            pl.BlockSpec((_BR, F_OUT), lambda i: (i, 0)),
            pl.BlockSpec((NC, _BR, F_HID), lambda i: (0, i, 0)),
            pl.BlockSpec((NC, _BR, 1), lambda i: (0, i, 0)),
            pl.BlockSpec((F_HID, F_OUT), lambda i: (0, 0)),
        ],
        out_specs=pl.BlockSpec((_BR, F_OUT), lambda i: (i, 0)),
        out_shape=jax.ShapeDtypeStruct((N, F_OUT), jnp.float32),
    )(hs, agg, deg2, W_neigh)


def kernel(x, edge_index, W_self1, W_neigh1, b1, W_self2, W_neigh2, b2):
    src = edge_index[0].astype(jnp.int32)
    dst = edge_index[1].astype(jnp.int32)

    zb1 = jnp.zeros((F_HID,), jnp.float32)
    xn1 = _tc_mm(x, W_neigh1, zb1)
    agg1, deg = _sc_agg_deg(xn1, src, dst)
    xs1 = _tc_mm(x, W_self1, b1)
    deg2 = deg.reshape(NC, N, 1)
    h = _tc_relu_comb(xs1, agg1, deg2)
    agg2 = _sc_agg(h, src, dst)
    hs2 = _tc_mm(h, W_self2, b2)
    return _tc_combine(hs2, agg2, deg2, W_neigh2)


# speedup vs baseline: 1.0545x; 1.0030x over previous
import jax
import jax.numpy as jnp
from jax import lax
from jax.experimental import pallas as pl
from jax.experimental.pallas import tpu as pltpu
from jax.experimental.pallas import tpu_sc as plsc

N = 10000
E = 320000
F_IN = 128
F_HID = 128
F_OUT = 64

NC = 2
NS = 16
NW = NC * NS
EPT = E // NW
CH = 64
NFULL = EPT // CH
REM_E = EPT - NFULL * CH
NSETS = 4
NQUAD = NFULL // NSETS
NMAIN = NQUAD * NSETS
RPT = 624
R_TAIL0 = NS * RPT
R_TAIL = N - R_TAIL0
ZFULL = RPT // CH
ZREM = RPT - ZFULL * CH


def _make_sc_agg(D, with_deg):
    out_type = jax.ShapeDtypeStruct((NC, N, D), jnp.float32)
    if with_deg:
        out_type = (out_type, jax.ShapeDtypeStruct((NC * N,), jnp.float32))
    scratch = [
        pltpu.VMEM_SHARED((N, D), jnp.float32),
        pltpu.VMEM((EPT,), jnp.int32),
        pltpu.VMEM((REM_E,), jnp.int32),
        pltpu.VMEM((REM_E, D), jnp.float32),
        pltpu.SemaphoreType.DMA,
    ]
    for _ in range(NSETS):
        scratch += [
            pltpu.VMEM((CH,), jnp.int32),
            pltpu.VMEM((CH, D), jnp.float32),
            pltpu.SemaphoreType.DMA,
            pltpu.SemaphoreType.DMA,
            pltpu.SemaphoreType.DMA,
            pltpu.SemaphoreType.DMA,
        ]
    if with_deg:
        scratch += [
            pltpu.VMEM_SHARED((N,), jnp.float32),
            pltpu.VMEM((CH,), jnp.float32),
            pltpu.VMEM((RPT,), jnp.float32),
        ]

    def body(feat, src, dst, *refs):
        if with_deg:
            out, deg_out = refs[0], refs[1]
            rest = refs[2:]
        else:
            out = refs[0]
            rest = refs[1:]
        acc_sh, src_all, dst_rem, rows_rem, rsem = rest[:5]
        sets = [tuple(rest[5 + 6 * q: 5 + 6 * (q + 1)]) for q in range(NSETS)]
        if with_deg:
            deg_sh, ones_v, degrow_v = rest[5 + 6 * NSETS:]
        rows0 = sets[0][1]
        c = lax.axis_index("c")
        s = lax.axis_index("s")
        wid = c * NS + s
        r0 = pl.multiple_of(s * RPT, 8)
        ebase = pl.multiple_of(wid * EPT, 8)

        zv = jnp.zeros((16,), jnp.float32)

        def zrow(i, carry):
            for j in range(D // 16):
                rows0[i, pl.ds(j * 16, 16)] = zv
            return carry

        lax.fori_loop(0, CH, zrow, 0)
        zdescs = []
        for k in range(ZFULL):
            zdescs.append(pltpu.async_copy(
                rows0, acc_sh.at[pl.ds(r0 + k * CH, CH)], sets[k % NSETS][4]))
        zdescs.append(pltpu.async_copy(
            rows0.at[pl.ds(0, ZREM)],
            acc_sh.at[pl.ds(r0 + ZFULL * CH, ZREM)], sets[ZFULL % NSETS][4]))
        if with_deg:
            for j in range(CH // 16):
                ones_v[pl.ds(j * 16, 16)] = jnp.ones((16,), jnp.float32)
            for j in range(RPT // 16):
                degrow_v[pl.ds(j * 16, 16)] = zv
            zdescs.append(pltpu.async_copy(
                degrow_v, deg_sh.at[pl.ds(r0, RPT)], sets[0][5]))
        pltpu.sync_copy(src.at[pl.ds(ebase, EPT)], src_all)
        for d in zdescs:
            d.wait()

        @pl.when(s == NS - 1)
        def _zero_tail():
            pltpu.sync_copy(rows0.at[pl.ds(0, R_TAIL)],
                            acc_sh.at[pl.ds(R_TAIL0, R_TAIL)])
            if with_deg:
                pltpu.sync_copy(degrow_v.at[pl.ds(0, R_TAIL)],
                                deg_sh.at[pl.ds(R_TAIL0, R_TAIL)])

        plsc.subcore_barrier()

        def src_sl(i):
            return src_all.at[pl.ds(pl.multiple_of(i * CH, 8), CH)]

        def start_chunk(i, q):
            dst_v, rows, isem, gsem = sets[q][0], sets[q][1], sets[q][2], sets[q][3]
            pltpu.async_copy(dst.at[pl.ds(pl.multiple_of(ebase + i * CH, 8), CH)],
                             dst_v, isem)
            pltpu.async_copy(feat.at[src_sl(i)], rows, gsem)

        def start_scatter(q):
            dst_v, rows, ssem, dsem = sets[q][0], sets[q][1], sets[q][4], sets[q][5]
            pltpu.async_copy(rows, acc_sh.at[dst_v], ssem, add=True)
            if with_deg:
                pltpu.async_copy(ones_v, deg_sh.at[dst_v], dsem, add=True)

        def wait_idx(q):
            dst_v, isem = sets[q][0], sets[q][2]
            pltpu.make_async_copy(dst.at[pl.ds(ebase, CH)], dst_v, isem).wait()

        def wait_gather(q):
            rows, gsem = sets[q][1], sets[q][3]
            pltpu.make_async_copy(feat.at[src_sl(0)], rows, gsem).wait()

        def wait_scatter(q):
            dst_v, rows, ssem, dsem = sets[q][0], sets[q][1], sets[q][4], sets[q][5]
            pltpu.make_async_copy(rows, acc_sh.at[dst_v], ssem).wait()
            if with_deg:
                pltpu.make_async_copy(ones_v, deg_sh.at[dst_v], dsem).wait()

        rem_lo = pl.multiple_of(NFULL * CH, 8)
        pltpu.sync_copy(dst.at[pl.ds(ebase + rem_lo, REM_E)], dst_rem)
        g_rem = pltpu.async_copy(feat.at[src_all.at[pl.ds(rem_lo, REM_E)]],
                                 rows_rem, rsem)
        for t, i in enumerate(range(NMAIN, NFULL)):
            start_chunk(i, t)
        for t, i in enumerate(range(NMAIN, NFULL)):
            wait_gather(t)
            wait_idx(t)
            start_scatter(t)
        for t, i in enumerate(range(NMAIN, NFULL)):
            wait_scatter(t)

        for q in range(NSETS):
            start_chunk(q, q)

        def quad(g, carry):
            i = NSETS * g
            for q in range(NSETS):
                wait_gather(q)
                wait_idx(q)
                start_scatter(q)
            for q in range(NSETS):
                nq = jnp.where(i + q + NSETS < NMAIN, i + q + NSETS, 0)
                wait_scatter(q)
                start_chunk(nq, q)
            return carry

        lax.fori_loop(0, NQUAD, quad, 0)
        for q in range(NSETS):
            wait_idx(q)
            wait_gather(q)
        g_rem.wait()
        d_rem = pltpu.async_copy(rows_rem, acc_sh.at[dst_rem],
                                 sets[0][4], add=True)
        if with_deg:
            pltpu.async_copy(ones_v.at[pl.ds(0, REM_E)],
                             deg_sh.at[dst_rem], sets[0][5], add=True).wait()
        d_rem.wait()
        plsc.subcore_barrier()

        def wr_start(lo, nrows, q):
            rows, ssem = sets[q][1], sets[q][4]
            pltpu.sync_copy(acc_sh.at[pl.ds(lo, nrows)],
                            rows.at[pl.ds(0, nrows)])
            pltpu.async_copy(rows.at[pl.ds(0, nrows)],
                             out.at[c, pl.ds(lo, nrows)], ssem)

        def wr_wait(lo, nrows, q):
            rows, ssem = sets[q][1], sets[q][4]
            pltpu.make_async_copy(rows.at[pl.ds(0, nrows)],
                                  out.at[c, pl.ds(lo, nrows)], ssem).wait()

        lo_rem = pl.multiple_of(r0 + ZFULL * CH, 8)
        outstanding = {}
        for k in range(ZFULL):
            q = k % NSETS
            if q in outstanding:
                wr_wait(lo_rem, outstanding.pop(q), q)
            wr_start(pl.multiple_of(r0 + k * CH, 8), CH, q)
            outstanding[q] = CH
        q = ZFULL % NSETS
        if q in outstanding:
            wr_wait(lo_rem, outstanding.pop(q), q)
        wr_start(lo_rem, ZREM, q)
        outstanding[q] = ZREM
        for q2, nr in outstanding.items():
            wr_wait(lo_rem, nr, q2)
        if with_deg:
            pltpu.sync_copy(deg_sh.at[pl.ds(r0, RPT)], degrow_v)
            d0 = pl.multiple_of(c * N + r0, 8)
            pltpu.sync_copy(degrow_v, deg_out.at[pl.ds(d0, RPT)])

        @pl.when(s == NS - 1)
        def _write_tail():
            pltpu.sync_copy(acc_sh.at[pl.ds(R_TAIL0, R_TAIL)],
                            rows0.at[pl.ds(0, R_TAIL)])
            pltpu.sync_copy(rows0.at[pl.ds(0, R_TAIL)],
                            out.at[c, pl.ds(R_TAIL0, R_TAIL)])
            if with_deg:
                pltpu.sync_copy(deg_sh.at[pl.ds(R_TAIL0, R_TAIL)],
                                degrow_v.at[pl.ds(0, R_TAIL)])
                dt = pl.multiple_of(c * N + R_TAIL0, 8)
                pltpu.sync_copy(degrow_v.at[pl.ds(0, R_TAIL)],
                                deg_out.at[pl.ds(dt, R_TAIL)])

    return pl.kernel(
        body,
        out_type=out_type,
        mesh=plsc.VectorSubcoreMesh(core_axis_name="c", subcore_axis_name="s"),
        scratch_types=scratch,
        name=f"sc_agg_d{D}" + ("_deg" if with_deg else ""),
    )


_sc_agg_deg = _make_sc_agg(F_HID, with_deg=True)
_sc_agg = _make_sc_agg(F_HID, with_deg=False)


_BR = 5000


def _mm2_body(x_ref, wn_ref, ws_ref, b_ref, on_ref, os_ref):
    xb = x_ref[...]
    on_ref[...] = jnp.dot(xb, wn_ref[...], preferred_element_type=jnp.float32)
    os_ref[...] = jnp.dot(xb, ws_ref[...], preferred_element_type=jnp.float32) + b_ref[...]


def _tc_mm2(x, W_neigh, W_self, b):
    return pl.pallas_call(
        _mm2_body,
        grid=(N // _BR,),
        in_specs=[
            pl.BlockSpec((_BR, F_IN), lambda i: (i, 0)),
            pl.BlockSpec((F_IN, F_HID), lambda i: (0, 0)),
            pl.BlockSpec((F_IN, F_HID), lambda i: (0, 0)),
            pl.BlockSpec((1, F_HID), lambda i: (0, 0)),
        ],
        out_specs=[
            pl.BlockSpec((_BR, F_HID), lambda i: (i, 0)),
            pl.BlockSpec((_BR, F_HID), lambda i: (i, 0)),
        ],
        out_shape=[
            jax.ShapeDtypeStruct((N, F_HID), jnp.float32),
            jax.ShapeDtypeStruct((N, F_HID), jnp.float32),
        ],
    )(x, W_neigh, W_self, b.reshape(1, -1))


def _mm_body(x_ref, w_ref, b_ref, o_ref):
    o_ref[...] = jnp.dot(x_ref[...], w_ref[...],
                         preferred_element_type=jnp.float32) + b_ref[...]


def _tc_mm(x, W, b):
    Din, Dout = W.shape
    return pl.pallas_call(
        _mm_body,
        grid=(N // _BR,),
        in_specs=[
            pl.BlockSpec((_BR, Din), lambda i: (i, 0)),
            pl.BlockSpec((Din, Dout), lambda i: (0, 0)),
            pl.BlockSpec((1, Dout), lambda i: (0, 0)),
        ],
        out_specs=pl.BlockSpec((_BR, Dout), lambda i: (i, 0)),
        out_shape=jax.ShapeDtypeStruct((N, Dout), jnp.float32),
    )(x, W, b.reshape(1, -1))


def _relu_comb_body(xs_ref, agg_ref, deg_ref, h_ref):
    deg = jnp.maximum(deg_ref[0] + deg_ref[1], 1.0)
    h_ref[...] = jnp.maximum(xs_ref[...] + (agg_ref[0] + agg_ref[1]) / deg, 0.0)


def _tc_relu_comb(xs, agg, deg2):
    return pl.pallas_call(
        _relu_comb_body,
        grid=(N // _BR,),
        in_specs=[
            pl.BlockSpec((_BR, F_HID), lambda i: (i, 0)),
            pl.BlockSpec((NC, _BR, F_HID), lambda i: (0, i, 0)),
            pl.BlockSpec((NC, _BR, 1), lambda i: (0, i, 0)),
        ],
        out_specs=pl.BlockSpec((_BR, F_HID), lambda i: (i, 0)),
        out_shape=jax.ShapeDtypeStruct((N, F_HID), jnp.float32),
    )(xs, agg, deg2)


def _comb_body(hs_ref, agg_ref, deg_ref, wn_ref, out_ref):
    deg = jnp.maximum(deg_ref[0] + deg_ref[1], 1.0)
    h_neigh = (agg_ref[0] + agg_ref[1]) / deg
    out_ref[...] = hs_ref[...] + jnp.dot(
        h_neigh, wn_ref[...], preferred_element_type=jnp.float32)


def _tc_combine(hs, agg, deg2, W_neigh):
    return pl.pallas_call(
        _comb_body,
        grid=(N // _BR,),
        in_specs=[
            pl.BlockSpec((_BR, F_OUT), lambda i: (i, 0)),
            pl.BlockSpec((NC, _BR, F_HID), lambda i: (0, i, 0)),
            pl.BlockSpec((NC, _BR, 1), lambda i: (0, i, 0)),
            pl.BlockSpec((F_HID, F_OUT), lambda i: (0, 0)),
        ],
        out_specs=pl.BlockSpec((_BR, F_OUT), lambda i: (i, 0)),
        out_shape=jax.ShapeDtypeStruct((N, F_OUT), jnp.float32),
    )(hs, agg, deg2, W_neigh)


def kernel(x, edge_index, W_self1, W_neigh1, b1, W_self2, W_neigh2, b2):
    src = edge_index[0].astype(jnp.int32)
    dst = edge_index[1].astype(jnp.int32)

    zb1 = jnp.zeros((F_HID,), jnp.float32)
    xn1 = _tc_mm(x, W_neigh1, zb1)
    agg1, deg = _sc_agg_deg(xn1, src, dst)
    xs1 = _tc_mm(x, W_self1, b1)
    deg2 = deg.reshape(NC, N, 1)
    h = _tc_relu_comb(xs1, agg1, deg2)
    agg2 = _sc_agg(h, src, dst)
    hs2 = _tc_mm(h, W_self2, b2)
    return _tc_combine(hs2, agg2, deg2, W_neigh2)
